# baseline (speedup 1.0000x reference)
import functools

import jax
import jax.numpy as jnp
from jax import lax
from jax.experimental import pallas as pl
from jax.experimental.pallas import tpu as pltpu
from jax.experimental.pallas import tpu_sc as plsc

LANES = 16
NSUB = 16
NCORE = 2
ROWPAD = 128


def _pool_kernel(B, L, E, V, C):
    b_per_sub = B // NSUB
    n_chunks = b_per_sub // C
    n_vregs = E // LANES
    inv_l = 1.0 / L
    blk = 128
    n_full_blocks = V // blk
    rem = V - n_full_blocks * blk
    v_pad = (n_full_blocks + (1 if rem else 0)) * blk
    max_blocks_per_sub = (n_full_blocks + NSUB - 1) // NSUB

    mesh = plsc.VectorSubcoreMesh(
        core_axis_name="c", subcore_axis_name="s",
        num_cores=NCORE, num_subcores=NSUB)

    @functools.partial(
        pl.kernel,
        out_type=(
            jax.ShapeDtypeStruct((B, ROWPAD), jnp.float32),
            jax.ShapeDtypeStruct((B, ROWPAD), jnp.float32),
            jax.ShapeDtypeStruct((v_pad, ROWPAD), jnp.float32),
            jax.ShapeDtypeStruct((v_pad, ROWPAD), jnp.float32),
        ),
        mesh=mesh,
        scratch_types=[
            pltpu.VMEM((E, blk), jnp.float32),
            pltpu.VMEM((blk, ROWPAD), jnp.float32),
            pltpu.VMEM((C, L), jnp.int32),
            pltpu.VMEM((2, C // 2, L, ROWPAD), jnp.float32),
            pltpu.VMEM((C, ROWPAD), jnp.float32),
            pltpu.SemaphoreType.DMA,
            pltpu.SemaphoreType.DMA,
        ],
        compiler_params=pltpu.CompilerParams(
            use_tc_tiling_on_sc=True, needs_layout_passes=False),
    )
    def pool(user_t, recipe_t, u_tail, r_tail, user_ing, recipe_ing,
             out_u, out_r, s_u, s_r,
             slab_v, blk_v, idx_v, rows_v, out_v, sem_a, sem_b):
        cid = lax.axis_index("c")
        sid = lax.axis_index("s")

        def transpose_block(src_ref, nrows):
            @pl.loop(0, nrows)
            def _(r):
                col = jnp.full((LANES,), r, jnp.int32)
                for k in range(n_vregs):
                    vals = plsc.load_gather(
                        src_ref,
                        [lax.iota(jnp.int32, LANES) + k * LANES, col])
                    blk_v[r, pl.ds(k * LANES, LANES)] = vals

        def phase1(t_ref, tail_ref, s_ref):
            @pl.loop(0, max_blocks_per_sub)
            def _(j):
                ib = sid + j * NSUB

                @pl.when(ib < n_full_blocks)
                def _():
                    off = pl.multiple_of(ib * blk, blk)
                    pltpu.sync_copy(
                        t_ref.at[pl.ds(0, E), pl.ds(off, blk)],
                        slab_v)
                    transpose_block(slab_v, blk)
                    pltpu.sync_copy(
                        blk_v, s_ref.at[pl.ds(off, blk)])

            @pl.when(sid == 0)
            def _():
                pltpu.sync_copy(tail_ref, slab_v)
                transpose_block(slab_v, rem)
                pltpu.sync_copy(
                    blk_v, s_ref.at[pl.ds(n_full_blocks * blk, blk)])

        def phase2(ing_ref, s_ref, pooled_ref):
            wbase = sid * b_per_sub
            half = C // 2
            sems = (sem_a, sem_b)
            zero = jnp.zeros((LANES,), jnp.float32)
            for p0 in range(C):
                for k0 in range(n_vregs, ROWPAD // LANES):
                    out_v[p0, pl.ds(k0 * LANES, LANES)] = zero

            def accum(h, p, out_p):
                zero = jnp.zeros((LANES,), jnp.float32)
                rv = rows_v.at[h, p]

                @pl.loop(0, L // 5,
                         init_carry=tuple(zero for _ in range(n_vregs)))
                def accs(r5, acc):
                    r = r5 * 5
                    for dr in range(5):
                        acc = tuple(
                            a + rv[r + dr, pl.ds(k * LANES, LANES)]
                            for k, a in enumerate(acc))
                    return acc

                for k in range(n_vregs):
                    out_v[out_p, pl.ds(k * LANES, LANES)] = accs[k] * inv_l

            @pl.loop(0, n_chunks)
            def _(ch):
                base = pl.multiple_of(wbase + ch * C, C)
                pltpu.sync_copy(ing_ref.at[pl.ds(base, C)], idx_v)
                for h in range(2):
                    for p in range(half):
                        pltpu.async_copy(
                            s_ref.at[idx_v.at[h * half + p]],
                            rows_v.at[h, p], sems[h])
                for h in range(2):
                    for p in range(half):
                        pltpu.make_async_copy(
                            s_ref.at[idx_v.at[h * half + p]],
                            rows_v.at[h, p], sems[h]).wait()
                    for p in range(half):
                        accum(h, p, h * half + p)
                pltpu.sync_copy(out_v, pooled_ref.at[pl.ds(base, C)])

        @pl.when(cid == 0)
        def _():
            phase1(user_t, u_tail, s_u)
            plsc.subcore_barrier()
            phase2(user_ing, s_u, out_u)

        @pl.when(cid == 1)
        def _():
            phase1(recipe_t, r_tail, s_r)
            plsc.subcore_barrier()
            phase2(recipe_ing, s_r, out_r)

    return pool


def _mlp_kernel(B, E, H1, H2, BLK):
    def body(xu_ref, xr_ref, w1u_ref, w1r_ref, b1_ref, w2_ref, b2_ref,
             w3_ref, b3_ref, o_ref):
        h = jnp.dot(xu_ref[...], w1u_ref[...],
                    preferred_element_type=jnp.float32)
        h = h + jnp.dot(xr_ref[...], w1r_ref[...],
                        preferred_element_type=jnp.float32)
        h = jnp.maximum(h + b1_ref[...], 0.0)
        h = jnp.dot(h, w2_ref[...], preferred_element_type=jnp.float32)
        h = jnp.maximum(h + b2_ref[...], 0.0)
        o = jnp.sum(h * w3_ref[...], axis=1, keepdims=True)
        o_ref[...] = o + b3_ref[...]

    grid = (B // BLK,)
    return pl.pallas_call(
        body,
        grid=grid,
        in_specs=[
            pl.BlockSpec((BLK, 2 * E), lambda i: (i, 0)),
            pl.BlockSpec((BLK, 2 * E), lambda i: (i, 0)),
            pl.BlockSpec((2 * E, H1), lambda i: (0, 0)),
            pl.BlockSpec((2 * E, H1), lambda i: (0, 0)),
            pl.BlockSpec((1, H1), lambda i: (0, 0)),
            pl.BlockSpec((H1, H2), lambda i: (0, 0)),
            pl.BlockSpec((1, H2), lambda i: (0, 0)),
            pl.BlockSpec((1, H2), lambda i: (0, 0)),
            pl.BlockSpec((1, 1), lambda i: (0, 0)),
        ],
        out_specs=pl.BlockSpec((BLK, 1), lambda i: (i, 0)),
        out_shape=jax.ShapeDtypeStruct((B, 1), jnp.float32),
    )


def kernel(user_ing, recipe_ing, user_table, recipe_table, W1, b1, W2, b2, W3, b3):
    B, L = user_ing.shape
    V, E = user_table.shape
    H1 = W1.shape[0]
    H2 = W2.shape[0]

    blk = 128
    tail0 = V - (V % blk)
    tail_pad = ((0, 0), (0, blk - (V - tail0)))
    pooled_u, pooled_r, _, _ = _pool_kernel(B, L, E, V, C=8)(
        user_table.T, recipe_table.T,
        jnp.pad(user_table[tail0:].T, tail_pad),
        jnp.pad(recipe_table[tail0:].T, tail_pad),
        user_ing, recipe_ing)

    W1t = W1.T
    zpad = jnp.zeros((E, H1), jnp.float32)
    out = _mlp_kernel(B, E, H1, H2, BLK=1024)(
        pooled_u, pooled_r,
        jnp.concatenate([W1t[:E], zpad], axis=0),
        jnp.concatenate([W1t[E:], zpad], axis=0),
        b1.reshape(1, H1),
        W2.T, b2.reshape(1, H2),
        W3.reshape(1, H2), b3.reshape(1, 1),
    )
    return out[:, 0]


# speedup vs baseline: 3.7992x; 3.7992x over previous
import functools

import jax
import jax.numpy as jnp
from jax import lax
from jax.experimental import pallas as pl
from jax.experimental.pallas import tpu as pltpu
from jax.experimental.pallas import tpu_sc as plsc

LANES = 16
NSUB = 16
NCORE = 2
ROWPAD = 128


def _repack_kernel(V, E, BI):
    def body(u_ref, r_ref, o_ref):
        u = u_ref[...]
        r = r_ref[...]
        o_ref[...] = jnp.concatenate([u.T, r.T], axis=1)

    grid = ((V + BI - 1) // BI,)
    return pl.pallas_call(
        body,
        grid=grid,
        in_specs=[
            pl.BlockSpec((E, BI), lambda i: (0, i)),
            pl.BlockSpec((E, BI), lambda i: (0, i)),
        ],
        out_specs=pl.BlockSpec((BI, 2 * E), lambda i: (i, 0)),
        out_shape=jax.ShapeDtypeStruct((V, 2 * E), jnp.float32),
    )


def _pool_kernel(B, L, E, V, C):
    b_per_sub = B // NSUB
    n_chunks = b_per_sub // C
    n_vregs = E // LANES
    inv_l = 1.0 / L

    mesh = plsc.VectorSubcoreMesh(
        core_axis_name="c", subcore_axis_name="s",
        num_cores=NCORE, num_subcores=NSUB)

    @functools.partial(
        pl.kernel,
        out_type=(
            jax.ShapeDtypeStruct((B, ROWPAD), jnp.float32),
            jax.ShapeDtypeStruct((B, ROWPAD), jnp.float32),
        ),
        mesh=mesh,
        scratch_types=[
            pltpu.VMEM((C, L), jnp.int32),
            pltpu.VMEM((2, C // 2, L, ROWPAD), jnp.float32),
            pltpu.VMEM((C, ROWPAD), jnp.float32),
            pltpu.SemaphoreType.DMA,
            pltpu.SemaphoreType.DMA,
        ],
        compiler_params=pltpu.CompilerParams(
            use_tc_tiling_on_sc=True, needs_layout_passes=False),
    )
    def pool(scratch, user_ing, recipe_ing, out_u, out_r,
             idx_v, rows_v, out_v, sem_a, sem_b):
        cid = lax.axis_index("c")
        sid = lax.axis_index("s")

        def phase2(ing_ref, pooled_ref, col_off):
            wbase = sid * b_per_sub
            half = C // 2
            sems = (sem_a, sem_b)
            zero = jnp.zeros((LANES,), jnp.float32)
            for p0 in range(C):
                for k0 in range(n_vregs, ROWPAD // LANES):
                    out_v[p0, pl.ds(k0 * LANES, LANES)] = zero

            def accum(h, p, out_p):
                rv = rows_v.at[h, p]

                @pl.loop(0, L // 5,
                         init_carry=tuple(zero for _ in range(n_vregs)))
                def accs(r5, acc):
                    r = r5 * 5
                    for dr in range(5):
                        acc = tuple(
                            a + rv[r + dr,
                                   pl.ds(col_off + k * LANES, LANES)]
                            for k, a in enumerate(acc))
                    return acc

                for k in range(n_vregs):
                    out_v[out_p, pl.ds(k * LANES, LANES)] = accs[k] * inv_l

            @pl.loop(0, n_chunks)
            def _(ch):
                base = pl.multiple_of(wbase + ch * C, C)
                pltpu.sync_copy(ing_ref.at[pl.ds(base, C)], idx_v)
                for h in range(2):
                    for p in range(half):
                        pltpu.async_copy(
                            scratch.at[idx_v.at[h * half + p]],
                            rows_v.at[h, p], sems[h])
                for h in range(2):
                    for p in range(half):
                        pltpu.make_async_copy(
                            scratch.at[idx_v.at[h * half + p]],
                            rows_v.at[h, p], sems[h]).wait()
                    for p in range(half):
                        accum(h, p, h * half + p)
                pltpu.sync_copy(out_v, pooled_ref.at[pl.ds(base, C)])

        @pl.when(cid == 0)
        def _():
            phase2(user_ing, out_u, 0)

        @pl.when(cid == 1)
        def _():
            phase2(recipe_ing, out_r, E)

    return pool


def _mlp_kernel(B, E, H1, H2, BLK):
    def body(xu_ref, xr_ref, w1u_ref, w1r_ref, b1_ref, w2_ref, b2_ref,
             w3_ref, b3_ref, o_ref):
        h = jnp.dot(xu_ref[...], w1u_ref[...],
                    preferred_element_type=jnp.float32)
        h = h + jnp.dot(xr_ref[...], w1r_ref[...],
                        preferred_element_type=jnp.float32)
        h = jnp.maximum(h + b1_ref[...], 0.0)
        h = jnp.dot(h, w2_ref[...], preferred_element_type=jnp.float32)
        h = jnp.maximum(h + b2_ref[...], 0.0)
        o = jnp.sum(h * w3_ref[...], axis=1, keepdims=True)
        o_ref[...] = o + b3_ref[...]

    grid = (B // BLK,)
    return pl.pallas_call(
        body,
        grid=grid,
        in_specs=[
            pl.BlockSpec((BLK, 2 * E), lambda i: (i, 0)),
            pl.BlockSpec((BLK, 2 * E), lambda i: (i, 0)),
            pl.BlockSpec((2 * E, H1), lambda i: (0, 0)),
            pl.BlockSpec((2 * E, H1), lambda i: (0, 0)),
            pl.BlockSpec((1, H1), lambda i: (0, 0)),
            pl.BlockSpec((H1, H2), lambda i: (0, 0)),
            pl.BlockSpec((1, H2), lambda i: (0, 0)),
            pl.BlockSpec((1, H2), lambda i: (0, 0)),
            pl.BlockSpec((1, 1), lambda i: (0, 0)),
        ],
        out_specs=pl.BlockSpec((BLK, 1), lambda i: (i, 0)),
        out_shape=jax.ShapeDtypeStruct((B, 1), jnp.float32),
    )


def kernel(user_ing, recipe_ing, user_table, recipe_table, W1, b1, W2, b2, W3, b3):
    B, L = user_ing.shape
    V, E = user_table.shape
    H1 = W1.shape[0]
    H2 = W2.shape[0]

    scratch = _repack_kernel(V, E, BI=1024)(user_table.T, recipe_table.T)
    pooled_u, pooled_r = _pool_kernel(B, L, E, V, C=8)(
        scratch, user_ing, recipe_ing)

    W1t = W1.T
    zpad = jnp.zeros((E, H1), jnp.float32)
    out = _mlp_kernel(B, E, H1, H2, BLK=1024)(
        pooled_u, pooled_r,
        jnp.concatenate([W1t[:E], zpad], axis=0),
        jnp.concatenate([W1t[E:], zpad], axis=0),
        b1.reshape(1, H1),
        W2.T, b2.reshape(1, H2),
        W3.reshape(1, H2), b3.reshape(1, 1),
    )
    return out[:, 0]


# speedup vs baseline: 5.8867x; 1.5495x over previous
import functools

import jax
import jax.numpy as jnp
from jax import lax
from jax.experimental import pallas as pl
from jax.experimental.pallas import tpu as pltpu
from jax.experimental.pallas import tpu_sc as plsc

LANES = 16
NSUB = 16
NCORE = 2
ROWPAD = 128


def _repack_kernel(V, E, BI):
    def body(u_ref, r_ref, o_ref):
        u = u_ref[...]
        r = r_ref[...]
        o_ref[...] = jnp.concatenate([u.T, r.T], axis=1)

    grid = ((V + BI - 1) // BI,)
    return pl.pallas_call(
        body,
        grid=grid,
        in_specs=[
            pl.BlockSpec((E, BI), lambda i: (0, i)),
            pl.BlockSpec((E, BI), lambda i: (0, i)),
        ],
        out_specs=pl.BlockSpec((BI, 2 * E), lambda i: (i, 0)),
        out_shape=jax.ShapeDtypeStruct((V, 2 * E), jnp.float32),
    )


def _pool_kernel(B, L, E, V, C):
    b_per_sub = B // NSUB
    n_chunks = b_per_sub // C
    n_vregs = E // LANES
    inv_l = 1.0 / L

    mesh = plsc.VectorSubcoreMesh(
        core_axis_name="c", subcore_axis_name="s",
        num_cores=NCORE, num_subcores=NSUB)

    @functools.partial(
        pl.kernel,
        out_type=(
            jax.ShapeDtypeStruct((B, ROWPAD), jnp.float32),
            jax.ShapeDtypeStruct((B, ROWPAD), jnp.float32),
        ),
        mesh=mesh,
        scratch_types=[
            pltpu.VMEM((C, L), jnp.int32),
            pltpu.VMEM((2, C // 2, L, ROWPAD), jnp.float32),
            pltpu.VMEM((C, ROWPAD), jnp.float32),
            pltpu.SemaphoreType.DMA,
            pltpu.SemaphoreType.DMA,
        ],
        compiler_params=pltpu.CompilerParams(
            use_tc_tiling_on_sc=True, needs_layout_passes=False),
    )
    def pool(scratch, user_ing, recipe_ing, out_u, out_r,
             idx_v, rows_v, out_v, sem_a, sem_b):
        cid = lax.axis_index("c")
        sid = lax.axis_index("s")

        def phase2(ing_ref, pooled_ref, col_off):
            wbase = sid * b_per_sub
            half = C // 2
            sems = (sem_a, sem_b)
            zero = jnp.zeros((LANES,), jnp.float32)
            for p0 in range(C):
                for k0 in range(n_vregs, ROWPAD // LANES):
                    out_v[p0, pl.ds(k0 * LANES, LANES)] = zero

            def accum(h, p, out_p):
                rv = rows_v.at[h, p]

                @pl.loop(0, L // 5,
                         init_carry=tuple(zero for _ in range(n_vregs)))
                def accs(r5, acc):
                    r = r5 * 5
                    for dr in range(5):
                        acc = tuple(
                            a + rv[r + dr,
                                   pl.ds(col_off + k * LANES, LANES)]
                            for k, a in enumerate(acc))
                    return acc

                for k in range(n_vregs):
                    out_v[out_p, pl.ds(k * LANES, LANES)] = accs[k] * inv_l

            @pl.loop(0, n_chunks)
            def _(ch):
                base = pl.multiple_of(wbase + ch * C, C)
                pltpu.sync_copy(ing_ref.at[pl.ds(base, C)], idx_v)
                for h in range(2):
                    for p in range(half):
                        pltpu.async_copy(
                            scratch.at[idx_v.at[h * half + p]],
                            rows_v.at[h, p], sems[h])
                for h in range(2):
                    for p in range(half):
                        pltpu.make_async_copy(
                            scratch.at[idx_v.at[h * half + p]],
                            rows_v.at[h, p], sems[h]).wait()
                    for p in range(half):
                        accum(h, p, h * half + p)
                pltpu.sync_copy(out_v, pooled_ref.at[pl.ds(base, C)])

        @pl.when(cid == 0)
        def _():
            phase2(user_ing, out_u, 0)

        @pl.when(cid == 1)
        def _():
            phase2(recipe_ing, out_r, E)

    return pool


def _mlp_kernel(B, E, H1, H2, BLK):
    def body(xu_ref, xr_ref, w1u_ref, w1r_ref, b1_ref, w2_ref, b2_ref,
             w3_ref, b3_ref, o_ref):
        h = jnp.dot(xu_ref[...], w1u_ref[...],
                    preferred_element_type=jnp.float32)
        h = h + jnp.dot(xr_ref[...], w1r_ref[...],
                        preferred_element_type=jnp.float32)
        h = jnp.maximum(h + b1_ref[...], 0.0)
        h = jnp.dot(h, w2_ref[...], preferred_element_type=jnp.float32)
        h = jnp.maximum(h + b2_ref[...], 0.0)
        o = jnp.sum(h * w3_ref[...], axis=1, keepdims=True)
        o_ref[...] = o + b3_ref[...]

    grid = (B // BLK,)
    return pl.pallas_call(
        body,
        grid=grid,
        in_specs=[
            pl.BlockSpec((BLK, 2 * E), lambda i: (i, 0)),
            pl.BlockSpec((BLK, 2 * E), lambda i: (i, 0)),
            pl.BlockSpec((2 * E, H1), lambda i: (0, 0)),
            pl.BlockSpec((2 * E, H1), lambda i: (0, 0)),
            pl.BlockSpec((1, H1), lambda i: (0, 0)),
            pl.BlockSpec((H1, H2), lambda i: (0, 0)),
            pl.BlockSpec((1, H2), lambda i: (0, 0)),
            pl.BlockSpec((1, H2), lambda i: (0, 0)),
            pl.BlockSpec((1, 1), lambda i: (0, 0)),
        ],
        out_specs=pl.BlockSpec((BLK, 1), lambda i: (i, 0)),
        out_shape=jax.ShapeDtypeStruct((B, 1), jnp.float32),
    )


def kernel(user_ing, recipe_ing, user_table, recipe_table, W1, b1, W2, b2, W3, b3):
    B, L = user_ing.shape
    V, E = user_table.shape
    H1 = W1.shape[0]
    H2 = W2.shape[0]

    scratch = _repack_kernel(V, E, BI=4096)(user_table.T, recipe_table.T)
    pooled_u, pooled_r = _pool_kernel(B, L, E, V, C=8)(
        scratch, user_ing, recipe_ing)

    W1t = W1.T
    zpad = jnp.zeros((E, H1), jnp.float32)
    out = _mlp_kernel(B, E, H1, H2, BLK=1024)(
        pooled_u, pooled_r,
        jnp.concatenate([W1t[:E], zpad], axis=0),
        jnp.concatenate([W1t[E:], zpad], axis=0),
        b1.reshape(1, H1),
        W2.T, b2.reshape(1, H2),
        W3.reshape(1, H2), b3.reshape(1, 1),
    )
    return out[:, 0]


# speedup vs baseline: 6.5386x; 1.1107x over previous
import functools

import jax
import jax.numpy as jnp
from jax import lax
from jax.experimental import pallas as pl
from jax.experimental.pallas import tpu as pltpu
from jax.experimental.pallas import tpu_sc as plsc

LANES = 16
NSUB = 16
NCORE = 2
ROWPAD = 128


def _repack_kernel(V, E, BI):
    def body(u_ref, r_ref, o_ref):
        u = u_ref[...]
        r = r_ref[...]
        o_ref[...] = jnp.concatenate([u.T, r.T], axis=1)

    grid = ((V + BI - 1) // BI,)
    return pl.pallas_call(
        body,
        grid=grid,
        in_specs=[
            pl.BlockSpec((E, BI), lambda i: (0, i)),
            pl.BlockSpec((E, BI), lambda i: (0, i)),
        ],
        out_specs=pl.BlockSpec((BI, 2 * E), lambda i: (i, 0)),
        out_shape=jax.ShapeDtypeStruct((V, 2 * E), jnp.float32),
    )


def _pool_kernel(B, L, E, V, C):
    b_per_sub = B // NSUB
    n_chunks = b_per_sub // C
    n_vregs = E // LANES
    inv_l = 1.0 / L

    mesh = plsc.VectorSubcoreMesh(
        core_axis_name="c", subcore_axis_name="s",
        num_cores=NCORE, num_subcores=NSUB)

    @functools.partial(
        pl.kernel,
        out_type=(
            jax.ShapeDtypeStruct((B, ROWPAD), jnp.float32),
            jax.ShapeDtypeStruct((B, ROWPAD), jnp.float32),
        ),
        mesh=mesh,
        scratch_types=[
            pltpu.VMEM((C, L), jnp.int32),
            pltpu.VMEM((2, C // 2, L, ROWPAD), jnp.float32),
            pltpu.VMEM((C, ROWPAD), jnp.float32),
            pltpu.SemaphoreType.DMA,
            pltpu.SemaphoreType.DMA,
        ],
        compiler_params=pltpu.CompilerParams(
            use_tc_tiling_on_sc=True, needs_layout_passes=False),
    )
    def pool(scratch, user_ing, recipe_ing, out_u, out_r,
             idx_v, rows_v, out_v, sem_a, sem_b):
        cid = lax.axis_index("c")
        sid = lax.axis_index("s")

        def phase2(ing_ref, pooled_ref, col_off):
            wbase = sid * b_per_sub
            half = C // 2
            sems = (sem_a, sem_b)
            zero = jnp.zeros((LANES,), jnp.float32)
            for p0 in range(C):
                for k0 in range(n_vregs, ROWPAD // LANES):
                    out_v[p0, pl.ds(k0 * LANES, LANES)] = zero

            def accum(h, p, out_p):
                rv = rows_v.at[h, p]

                @pl.loop(0, L // 5,
                         init_carry=tuple(zero for _ in range(n_vregs)))
                def accs(r5, acc):
                    r = r5 * 5
                    for dr in range(5):
                        acc = tuple(
                            a + rv[r + dr,
                                   pl.ds(col_off + k * LANES, LANES)]
                            for k, a in enumerate(acc))
                    return acc

                for k in range(n_vregs):
                    out_v[out_p, pl.ds(k * LANES, LANES)] = accs[k] * inv_l

            @pl.loop(0, n_chunks)
            def _(ch):
                base = pl.multiple_of(wbase + ch * C, C)
                pltpu.sync_copy(ing_ref.at[pl.ds(base, C)], idx_v)
                for h in range(2):
                    for p in range(half):
                        pltpu.async_copy(
                            scratch.at[idx_v.at[h * half + p]],
                            rows_v.at[h, p], sems[h])
                for h in range(2):
                    for p in range(half):
                        pltpu.make_async_copy(
                            scratch.at[idx_v.at[h * half + p]],
                            rows_v.at[h, p], sems[h]).wait()
                    for p in range(half):
                        accum(h, p, h * half + p)
                pltpu.sync_copy(out_v, pooled_ref.at[pl.ds(base, C)])

        @pl.when(cid == 0)
        def _():
            phase2(user_ing, out_u, 0)

        @pl.when(cid == 1)
        def _():
            phase2(recipe_ing, out_r, E)

    return pool


def _mlp_kernel(B, E, H1, H2, BLK):
    def body(xu_ref, xr_ref, w1u_ref, w1r_ref, b1_ref, w2_ref, b2_ref,
             w3_ref, b3_ref, o_ref):
        h = jnp.dot(xu_ref[...], w1u_ref[...],
                    preferred_element_type=jnp.float32)
        h = h + jnp.dot(xr_ref[...], w1r_ref[...],
                        preferred_element_type=jnp.float32)
        h = jnp.maximum(h + b1_ref[...], 0.0)
        h = jnp.dot(h, w2_ref[...], preferred_element_type=jnp.float32)
        h = jnp.maximum(h + b2_ref[...], 0.0)
        o = jnp.sum(h * w3_ref[...], axis=1, keepdims=True)
        o_ref[...] = o + b3_ref[...]

    grid = (B // BLK,)
    return pl.pallas_call(
        body,
        grid=grid,
        in_specs=[
            pl.BlockSpec((BLK, 2 * E), lambda i: (i, 0)),
            pl.BlockSpec((BLK, 2 * E), lambda i: (i, 0)),
            pl.BlockSpec((2 * E, H1), lambda i: (0, 0)),
            pl.BlockSpec((2 * E, H1), lambda i: (0, 0)),
            pl.BlockSpec((1, H1), lambda i: (0, 0)),
            pl.BlockSpec((H1, H2), lambda i: (0, 0)),
            pl.BlockSpec((1, H2), lambda i: (0, 0)),
            pl.BlockSpec((1, H2), lambda i: (0, 0)),
            pl.BlockSpec((1, 1), lambda i: (0, 0)),
        ],
        out_specs=pl.BlockSpec((BLK, 1), lambda i: (i, 0)),
        out_shape=jax.ShapeDtypeStruct((B, 1), jnp.float32),
    )


def kernel(user_ing, recipe_ing, user_table, recipe_table, W1, b1, W2, b2, W3, b3):
    B, L = user_ing.shape
    V, E = user_table.shape
    H1 = W1.shape[0]
    H2 = W2.shape[0]

    scratch = _repack_kernel(V, E, BI=8192)(user_table.T, recipe_table.T)
    pooled_u, pooled_r = _pool_kernel(B, L, E, V, C=8)(
        scratch, user_ing, recipe_ing)

    W1t = W1.T
    zpad = jnp.zeros((E, H1), jnp.float32)
    out = _mlp_kernel(B, E, H1, H2, BLK=1024)(
        pooled_u, pooled_r,
        jnp.concatenate([W1t[:E], zpad], axis=0),
        jnp.concatenate([W1t[E:], zpad], axis=0),
        b1.reshape(1, H1),
        W2.T, b2.reshape(1, H2),
        W3.reshape(1, H2), b3.reshape(1, 1),
    )
    return out[:, 0]


# speedup vs baseline: 6.8601x; 1.0492x over previous
import functools

import jax
import jax.numpy as jnp
from jax import lax
from jax.experimental import pallas as pl
from jax.experimental.pallas import tpu as pltpu
from jax.experimental.pallas import tpu_sc as plsc

LANES = 16
NSUB = 16
NCORE = 2
ROWPAD = 128


def _repack_kernel(V, E, BI):
    def body(u_ref, r_ref, o_ref):
        u = u_ref[...]
        r = r_ref[...]
        o_ref[...] = jnp.concatenate([u.T, r.T], axis=1)

    grid = ((V + BI - 1) // BI,)
    return pl.pallas_call(
        body,
        grid=grid,
        in_specs=[
            pl.BlockSpec((E, BI), lambda i: (0, i)),
            pl.BlockSpec((E, BI), lambda i: (0, i)),
        ],
        out_specs=pl.BlockSpec((BI, 2 * E), lambda i: (i, 0)),
        out_shape=jax.ShapeDtypeStruct((V, 2 * E), jnp.float32),
    )


def _pool_kernel(B, L, E, V, C):
    b_per_sub = B // NSUB
    n_chunks = b_per_sub // C
    n_vregs = E // LANES
    inv_l = 1.0 / L

    mesh = plsc.VectorSubcoreMesh(
        core_axis_name="c", subcore_axis_name="s",
        num_cores=NCORE, num_subcores=NSUB)

    @functools.partial(
        pl.kernel,
        out_type=(
            jax.ShapeDtypeStruct((B, ROWPAD), jnp.float32),
            jax.ShapeDtypeStruct((B, ROWPAD), jnp.float32),
        ),
        mesh=mesh,
        scratch_types=[
            pltpu.VMEM((C, L), jnp.int32),
            pltpu.VMEM((2, C // 2, L, ROWPAD), jnp.float32),
            pltpu.VMEM((C, ROWPAD), jnp.float32),
            pltpu.SemaphoreType.DMA,
            pltpu.SemaphoreType.DMA,
        ],
        compiler_params=pltpu.CompilerParams(
            use_tc_tiling_on_sc=True, needs_layout_passes=False),
    )
    def pool(scratch, user_ing, recipe_ing, out_u, out_r,
             idx_v, rows_v, out_v, sem_a, sem_b):
        cid = lax.axis_index("c")
        sid = lax.axis_index("s")

        def phase2(ing_ref, pooled_ref, col_off):
            wbase = sid * b_per_sub
            half = C // 2
            sems = (sem_a, sem_b)
            zero = jnp.zeros((LANES,), jnp.float32)
            for p0 in range(C):
                for k0 in range(n_vregs, ROWPAD // LANES):
                    out_v[p0, pl.ds(k0 * LANES, LANES)] = zero

            def accum(h, p, out_p):
                rv = rows_v.at[h, p]

                @pl.loop(0, L // 5,
                         init_carry=tuple(zero for _ in range(n_vregs)))
                def accs(r5, acc):
                    r = r5 * 5
                    for dr in range(5):
                        acc = tuple(
                            a + rv[r + dr,
                                   pl.ds(col_off + k * LANES, LANES)]
                            for k, a in enumerate(acc))
                    return acc

                for k in range(n_vregs):
                    out_v[out_p, pl.ds(k * LANES, LANES)] = accs[k] * inv_l

            @pl.loop(0, n_chunks)
            def _(ch):
                base = pl.multiple_of(wbase + ch * C, C)
                pltpu.sync_copy(ing_ref.at[pl.ds(base, C)], idx_v)
                for h in range(2):
                    for p in range(half):
                        pltpu.async_copy(
                            scratch.at[idx_v.at[h * half + p]],
                            rows_v.at[h, p], sems[h])
                for h in range(2):
                    for p in range(half):
                        pltpu.make_async_copy(
                            scratch.at[idx_v.at[h * half + p]],
                            rows_v.at[h, p], sems[h]).wait()
                    for p in range(half):
                        accum(h, p, h * half + p)
                pltpu.sync_copy(out_v, pooled_ref.at[pl.ds(base, C)])

        @pl.when(cid == 0)
        def _():
            phase2(user_ing, out_u, 0)

        @pl.when(cid == 1)
        def _():
            phase2(recipe_ing, out_r, E)

    return pool


def _mlp_kernel(B, E, H1, H2, BLK):
    def body(xu_ref, xr_ref, w1u_ref, w1r_ref, b1_ref, w2_ref, b2_ref,
             w3_ref, b3_ref, o_ref):
        h = jnp.dot(xu_ref[...], w1u_ref[...],
                    preferred_element_type=jnp.float32)
        h = h + jnp.dot(xr_ref[...], w1r_ref[...],
                        preferred_element_type=jnp.float32)
        h = jnp.maximum(h + b1_ref[...], 0.0)
        h = jnp.dot(h, w2_ref[...], preferred_element_type=jnp.float32)
        h = jnp.maximum(h + b2_ref[...], 0.0)
        o = jnp.sum(h * w3_ref[...], axis=1, keepdims=True)
        o_ref[...] = o + b3_ref[...]

    grid = (B // BLK,)
    return pl.pallas_call(
        body,
        grid=grid,
        in_specs=[
            pl.BlockSpec((BLK, 2 * E), lambda i: (i, 0)),
            pl.BlockSpec((BLK, 2 * E), lambda i: (i, 0)),
            pl.BlockSpec((2 * E, H1), lambda i: (0, 0)),
            pl.BlockSpec((2 * E, H1), lambda i: (0, 0)),
            pl.BlockSpec((1, H1), lambda i: (0, 0)),
            pl.BlockSpec((H1, H2), lambda i: (0, 0)),
            pl.BlockSpec((1, H2), lambda i: (0, 0)),
            pl.BlockSpec((1, H2), lambda i: (0, 0)),
            pl.BlockSpec((1, 1), lambda i: (0, 0)),
        ],
        out_specs=pl.BlockSpec((BLK, 1), lambda i: (i, 0)),
        out_shape=jax.ShapeDtypeStruct((B, 1), jnp.float32),
    )


def kernel(user_ing, recipe_ing, user_table, recipe_table, W1, b1, W2, b2, W3, b3):
    B, L = user_ing.shape
    V, E = user_table.shape
    H1 = W1.shape[0]
    H2 = W2.shape[0]

    scratch = _repack_kernel(V, E, BI=16384)(user_table.T, recipe_table.T)
    pooled_u, pooled_r = _pool_kernel(B, L, E, V, C=8)(
        scratch, user_ing, recipe_ing)

    W1t = W1.T
    zpad = jnp.zeros((E, H1), jnp.float32)
    out = _mlp_kernel(B, E, H1, H2, BLK=1024)(
        pooled_u, pooled_r,
        jnp.concatenate([W1t[:E], zpad], axis=0),
        jnp.concatenate([W1t[E:], zpad], axis=0),
        b1.reshape(1, H1),
        W2.T, b2.reshape(1, H2),
        W3.reshape(1, H2), b3.reshape(1, 1),
    )
    return out[:, 0]
